# baseline (speedup 1.0000x reference)
import functools

import jax
import jax.numpy as jnp
from jax import lax
from jax.experimental import pallas as pl
from jax.experimental.pallas import tpu as pltpu
from jax.experimental.pallas import tpu_sc as plsc

D_MODEL = 64
SCALE = 8.0
LANES = 16

NC = 2
NS = 16
NW = NC * NS
CHUNK = 128
NBUF = 4


@functools.cache
def _build(B):
    rows_per_w = B // NW
    nch = rows_per_w // CHUNK
    ngrp = nch // NBUF

    mesh = plsc.VectorSubcoreMesh(core_axis_name="c", subcore_axis_name="s")

    scratch = [pltpu.VMEM((nch, CHUNK), jnp.int32)]
    scratch += [pltpu.VMEM((CHUNK, D_MODEL), jnp.float32) for _ in range(2 * NBUF)]
    scratch += [pltpu.SemaphoreType.DMA for _ in range(2 * NBUF)]

    @functools.partial(
        pl.kernel,
        mesh=mesh,
        out_type=jax.ShapeDtypeStruct((B, D_MODEL), jnp.float32),
        scratch_types=scratch,
        compiler_params=pltpu.CompilerParams(use_tc_tiling_on_sc=False),
    )
    def emb(x_hbm, table_hbm, out_hbm, idx_v, *rest):
        gbuf = rest[:NBUF]
        obuf = rest[NBUF:2 * NBUF]
        gsem = rest[2 * NBUF:3 * NBUF]
        osem = rest[3 * NBUF:4 * NBUF]

        wid = lax.axis_index("s") * NC + lax.axis_index("c")
        base = wid * rows_per_w

        pltpu.sync_copy(x_hbm.at[wid], idx_v)

        for b in range(NBUF):
            pltpu.async_copy(table_hbm.at[idx_v.at[b]], gbuf[b], gsem[b])

        def group(g, carry):
            for b in range(NBUF):
                c = g * NBUF + b

                pltpu.make_async_copy(
                    table_hbm.at[idx_v.at[c]], gbuf[b], gsem[b]).wait()

                @pl.when(g > 0)
                def _wait_out():
                    pltpu.make_async_copy(
                        obuf[b], out_hbm.at[pl.ds(base, CHUNK)], osem[b]).wait()

                def row(i, acc):
                    for j in range(D_MODEL // LANES):
                        sl = pl.ds(LANES * j, LANES)
                        obuf[b][i, sl] = gbuf[b][i, sl] * SCALE
                    return acc
                lax.fori_loop(0, CHUNK, row, 0, unroll=4)

                @pl.when(g < ngrp - 1)
                def _next_gather():
                    pltpu.async_copy(
                        table_hbm.at[idx_v.at[c + NBUF]], gbuf[b], gsem[b])

                pltpu.async_copy(
                    obuf[b], out_hbm.at[pl.ds(base + c * CHUNK, CHUNK)], osem[b])
            return carry

        lax.fori_loop(0, ngrp, group, 0)

        for b in range(NBUF):
            pltpu.make_async_copy(
                obuf[b], out_hbm.at[pl.ds(base, CHUNK)], osem[b]).wait()

    return emb


@jax.jit
def kernel(x, table):
    B = x.size
    x_r = x.reshape(NW, B // (NW * CHUNK), CHUNK).astype(jnp.int32)
    out = _build(B)(x_r, table)
    return out.reshape(*x.shape, D_MODEL)


# speedup vs baseline: 1.2162x; 1.2162x over previous
import functools

import jax
import jax.numpy as jnp
from jax import lax
from jax.experimental import pallas as pl
from jax.experimental.pallas import tpu as pltpu
from jax.experimental.pallas import tpu_sc as plsc

D_MODEL = 64
SCALE = 8.0
LANES = 16

NC = 2
NS = 16
NW = NC * NS
CHUNK = 64
NBUF = 4


@functools.cache
def _build(B):
    rows_per_w = B // NW
    nch = rows_per_w // CHUNK
    ngrp = nch // NBUF

    mesh = plsc.VectorSubcoreMesh(core_axis_name="c", subcore_axis_name="s")

    scratch = [pltpu.VMEM((nch, CHUNK), jnp.int32)]
    scratch += [pltpu.VMEM((CHUNK,), jnp.int32) for _ in range(NBUF)]
    scratch += [pltpu.VMEM((CHUNK, 128), jnp.float32) for _ in range(NBUF)]
    scratch += [pltpu.VMEM((CHUNK, D_MODEL), jnp.float32) for _ in range(NBUF)]
    scratch += [pltpu.SemaphoreType.DMA for _ in range(2 * NBUF)]

    @functools.partial(
        pl.kernel,
        mesh=mesh,
        out_type=jax.ShapeDtypeStruct((B, D_MODEL), jnp.float32),
        scratch_types=scratch,
    )
    def emb(x_hbm, table_hbm, out_hbm, idx_v, *rest):
        gidx = rest[:NBUF]
        gbuf = rest[NBUF:2 * NBUF]
        obuf = rest[2 * NBUF:3 * NBUF]
        gsem = rest[3 * NBUF:4 * NBUF]
        osem = rest[4 * NBUF:5 * NBUF]

        wid = lax.axis_index("s") * NC + lax.axis_index("c")
        base = wid * rows_per_w

        pltpu.sync_copy(x_hbm.at[wid], idx_v)

        def start_gather(c, b):
            def halve(v, _):
                sl = pl.ds(LANES * v, LANES)
                gidx[b][sl] = lax.shift_right_logical(idx_v[c, sl], 1)
                return _
            lax.fori_loop(0, CHUNK // LANES, halve, 0, unroll=True)
            pltpu.async_copy(table_hbm.at[gidx[b]], gbuf[b], gsem[b])

        for b in range(NBUF):
            start_gather(b, b)

        def group(g, carry):
            for b in range(NBUF):
                c = g * NBUF + b

                pltpu.make_async_copy(
                    table_hbm.at[gidx[b]], gbuf[b], gsem[b]).wait()

                @pl.when(g > 0)
                def _wait_out():
                    pltpu.make_async_copy(
                        obuf[b], out_hbm.at[pl.ds(base, CHUNK)], osem[b]).wait()

                def grp(gi, acc):
                    par = idx_v[c, pl.ds(gi * LANES, LANES)] & 1
                    for r in range(LANES):
                        i = gi * LANES + r
                        off = par[r] * D_MODEL
                        for j in range(D_MODEL // LANES):
                            obuf[b][i, pl.ds(LANES * j, LANES)] = (
                                gbuf[b][i, pl.ds(off + LANES * j, LANES)]
                                * SCALE)
                    return acc
                lax.fori_loop(0, CHUNK // LANES, grp, 0)

                @pl.when(g < ngrp - 1)
                def _next_gather():
                    start_gather(c + NBUF, b)

                pltpu.async_copy(
                    obuf[b], out_hbm.at[pl.ds(base + c * CHUNK, CHUNK)], osem[b])
            return carry

        lax.fori_loop(0, ngrp, group, 0)

        for b in range(NBUF):
            pltpu.make_async_copy(
                obuf[b], out_hbm.at[pl.ds(base, CHUNK)], osem[b]).wait()

    return emb


@jax.jit
def kernel(x, table):
    B = x.size
    x_r = x.reshape(NW, B // (NW * CHUNK), CHUNK).astype(jnp.int32)
    table2 = table.reshape(table.shape[0] // 2, 2 * table.shape[1])
    out = _build(B)(x_r, table2)
    return out.reshape(*x.shape, D_MODEL)
